# baseline (speedup 1.0000x reference)
import functools

import jax
import jax.numpy as jnp
from jax import lax
from jax.experimental import pallas as pl
from jax.experimental.pallas import tpu as pltpu
from jax.experimental.pallas import tpu_sc as plsc

_B, _L = 1024, 200
_EMB, _PDIM = 32, 32
_BL = _B * _L
_NW = 32
_PER_W = _BL // _NW
_GATHER = 128
_GPB = 10
_BUF = _GATHER * _GPB
_OUTER = _PER_W // _BUF
_POS_REP = 4
_POS_ROWS = _POS_REP * _L
_POS_WRITES = _PER_W // _POS_ROWS


def _body(ids_hbm, word_hbm, pos_hbm, out_hbm, idx_v, word_v, pos_v, gsem):
    c = lax.axis_index("c")
    s = lax.axis_index("s")
    wid = s * 2 + c
    base = wid * _PER_W

    for r in range(_POS_REP):
        pltpu.sync_copy(pos_hbm.at[pl.ds(0, _L)], pos_v.at[pl.ds(r * _L, _L)])
    for k in range(_POS_WRITES):
        pltpu.sync_copy(
            pos_v,
            out_hbm.at[pl.ds(base + k * _POS_ROWS, _POS_ROWS), pl.ds(_EMB, _PDIM)],
        )

    def outer(step, carry):
        pltpu.sync_copy(ids_hbm.at[wid, step], idx_v)
        copies = [
            pltpu.async_copy(
                word_hbm.at[idx_v.at[j]],
                word_v.at[pl.ds(j * _GATHER, _GATHER)],
                gsem,
            )
            for j in range(_GPB)
        ]
        for cp in copies:
            cp.wait()
        pltpu.sync_copy(
            word_v,
            out_hbm.at[pl.ds(base + step * _BUF, _BUF), pl.ds(0, _EMB)],
        )
        return carry

    lax.fori_loop(0, _OUTER, outer, 0)


@jax.jit
def _emb_lookup(ids, word_table, pos_table):
    mesh = plsc.VectorSubcoreMesh(core_axis_name="c", subcore_axis_name="s")
    f = pl.kernel(
        _body,
        mesh=mesh,
        compiler_params=pltpu.CompilerParams(use_tc_tiling_on_sc=False),
        out_type=jax.ShapeDtypeStruct((_BL, _EMB + _PDIM), jnp.float32),
        scratch_types=[
            pltpu.VMEM((_GPB, _GATHER), jnp.int32),
            pltpu.VMEM((_BUF, _EMB), jnp.float32),
            pltpu.VMEM((_POS_ROWS, _PDIM), jnp.float32),
            pltpu.SemaphoreType.DMA,
        ],
    )
    return f(ids, word_table, pos_table)


def kernel(input_ids, word_table, pos_table):
    ids = input_ids.reshape(_NW, _OUTER, _GPB, _GATHER)
    out = _emb_lookup(ids, word_table, pos_table)
    return out.reshape(_B, _L, _EMB + _PDIM)
